# baseline (speedup 1.0000x reference)
import functools

import jax
import jax.numpy as jnp
from jax import lax
from jax.experimental import pallas as pl
from jax.experimental.pallas import tpu as pltpu
from jax.experimental.pallas import tpu_sc as plsc

D_FEAT = 128
D_EDGE = 16
OUT_DIM = 128
_SC_BLOCK = 128


def _nodeproj_body(nf, ws, wd, ps, pd):
    x = nf[...]
    ps[...] = jnp.dot(x, ws[...],
                      preferred_element_type=jnp.float32).astype(jnp.bfloat16)
    pd[...] = jnp.dot(x, wd[...],
                      preferred_element_type=jnp.float32).astype(jnp.bfloat16)


def _node_projections(node_feat, W_s, W_d):
    N = node_feat.shape[0]
    BLK = 2000
    return pl.pallas_call(
        _nodeproj_body,
        grid=(N // BLK,),
        in_specs=[
            pl.BlockSpec((BLK, D_FEAT), lambda i: (i, 0)),
            pl.BlockSpec((D_FEAT, OUT_DIM), lambda i: (0, 0)),
            pl.BlockSpec((D_FEAT, OUT_DIM), lambda i: (0, 0)),
        ],
        out_specs=[
            pl.BlockSpec((BLK, OUT_DIM), lambda i: (i, 0)),
            pl.BlockSpec((BLK, OUT_DIM), lambda i: (i, 0)),
        ],
        out_shape=[
            jax.ShapeDtypeStruct((N, OUT_DIM), jnp.bfloat16),
            jax.ShapeDtypeStruct((N, OUT_DIM), jnp.bfloat16),
        ],
    )(node_feat, W_s, W_d)


def _make_gather_sum(E):
    info = plsc.get_sparse_core_info()
    NC, NS = info.num_cores, info.num_subcores
    NW = NC * NS
    B = _SC_BLOCK
    nblk = E // B
    mesh = plsc.VectorSubcoreMesh(core_axis_name="c", subcore_axis_name="s")

    PK = OUT_DIM // 2

    @functools.partial(
        pl.kernel,
        mesh=mesh,
        compiler_params=pltpu.CompilerParams(use_tc_tiling_on_sc=False),
        out_type=jax.ShapeDtypeStruct((E, PK), jnp.int32),
        scratch_types=[
            pltpu.VMEM((B,), jnp.int32),
            pltpu.VMEM((B,), jnp.int32),
            pltpu.VMEM((B, PK), jnp.int32),
            pltpu.VMEM((B, PK), jnp.int32),
            pltpu.SemaphoreType.DMA,
            pltpu.SemaphoreType.DMA,
        ],
    )
    def gather_sum(ps_hbm, pd_hbm, src_hbm, dst_hbm, g_hbm,
                   idx_s, idx_d, buf_s, buf_d, sem_s, sem_d):
        wid = lax.axis_index("s") * NC + lax.axis_index("c")
        my_n = (nblk - wid + NW - 1) // NW

        def blk_body(i, carry):
            base = (wid + i * NW) * B
            pltpu.sync_copy(src_hbm.at[pl.ds(base, B)], idx_s)
            pltpu.sync_copy(dst_hbm.at[pl.ds(base, B)], idx_d)
            c1 = pltpu.async_copy(ps_hbm.at[idx_s], buf_s, sem_s)
            c2 = pltpu.async_copy(pd_hbm.at[idx_d], buf_d, sem_d)
            c1.wait()
            c2.wait()

            hi_mask = jnp.int32(-65536)
            half = jnp.int32(0x8000)

            def row_body(r, rcarry):
                for c in range(PK // 16):
                    sl = pl.ds(c * 16, 16)
                    s = buf_s[r, sl]
                    d = buf_d[r, sl]
                    fl = (lax.bitcast_convert_type(s << 16, jnp.float32)
                          + lax.bitcast_convert_type(d << 16, jnp.float32))
                    fh = (lax.bitcast_convert_type(s & hi_mask, jnp.float32)
                          + lax.bitcast_convert_type(d & hi_mask, jnp.float32))
                    bl = lax.bitcast_convert_type(fl, jnp.int32) + half
                    bh = lax.bitcast_convert_type(fh, jnp.int32) + half
                    buf_s[r, sl] = (lax.shift_right_logical(bl, 16)
                                    | (bh & hi_mask))
                return rcarry

            lax.fori_loop(0, B, row_body, 0)
            pltpu.sync_copy(buf_s, g_hbm.at[pl.ds(base, B)])
            return carry

        lax.fori_loop(0, my_n, blk_body, 0)

    return gather_sum


def _edge_body(ea, we, bb, g, out):
    g32 = g[...]
    lo = jax.lax.bitcast_convert_type(g32 << 16, jnp.float32)
    hi = jax.lax.bitcast_convert_type(g32 & jnp.int32(-65536), jnp.float32)
    gf = jnp.concatenate([lo, hi], axis=-1)
    out[...] = (gf
                + jnp.dot(ea[...], we[...], preferred_element_type=jnp.float32)
                + bb[...])


def _edge_combine(edge_attr, W_e, b2d, G):
    E = edge_attr.shape[0]
    BLK = 4000
    return pl.pallas_call(
        _edge_body,
        grid=(E // BLK,),
        in_specs=[
            pl.BlockSpec((BLK, D_EDGE), lambda i: (i, 0)),
            pl.BlockSpec((D_EDGE, OUT_DIM), lambda i: (0, 0)),
            pl.BlockSpec((1, OUT_DIM), lambda i: (0, 0)),
            pl.BlockSpec((BLK, OUT_DIM // 2), lambda i: (i, 0)),
        ],
        out_specs=pl.BlockSpec((BLK, OUT_DIM), lambda i: (i, 0)),
        out_shape=jax.ShapeDtypeStruct((E, OUT_DIM), jnp.float32),
    )(edge_attr, W_e, b2d, G)


def _pack_cols(p):
    h = OUT_DIM // 2
    return jax.lax.bitcast_convert_type(
        jnp.stack([p[:, :h], p[:, h:]], axis=-1), jnp.int32)


def kernel(node_feat, edge_attr, edge_index, W, b):
    W_e = W[:D_EDGE]
    W_s = W[D_EDGE:D_EDGE + D_FEAT]
    W_d = W[D_EDGE + D_FEAT:]
    src = edge_index[0]
    dst = edge_index[1]
    ps, pd = _node_projections(node_feat, W_s, W_d)
    G = _make_gather_sum(edge_attr.shape[0])(_pack_cols(ps), _pack_cols(pd),
                                             src, dst)
    return _edge_combine(edge_attr, W_e, b.reshape(1, OUT_DIM), G)
